# baseline (speedup 1.0000x reference)
import functools

import jax
import jax.numpy as jnp
from jax import lax
from jax.experimental import pallas as pl
from jax.experimental.pallas import tpu as pltpu
from jax.experimental.pallas import tpu_sc as plsc

N = 10000
E = 320000
DI = 128
DH = 64
DL = 20
B = 8
LQ = 10

NC = 2
NS = 16
NW = NC * NS
K = 512
CH0 = 14
CH1 = 26
CHP = max(CH0, CH1)
NP = 10240
RPT = NP // NS
BLK = 1024
G = NP // BLK

_vmesh = plsc.VectorSubcoreMesh(core_axis_name="c", subcore_axis_name="s")
_sc_params = pltpu.CompilerParams(use_tc_tiling_on_sc=False)


def _make_spmm(DW):
    @functools.partial(
        pl.kernel,
        out_type=jax.ShapeDtypeStruct((NC, NP, DW), jnp.float32),
        mesh=_vmesh,
        scratch_types=[
            pltpu.VMEM((CHP, K), jnp.int32),
            pltpu.VMEM((CHP, K), jnp.int32),
            pltpu.VMEM((K, DW), jnp.float32),
            pltpu.VMEM_SHARED((NP, DW), jnp.float32),
        ],
        compiler_params=_sc_params,
    )
    def _k(hp_hbm, row_hbm, col_hbm, zero_hbm, out_hbm, rowv, colv, buf,
           accum):
        c = lax.axis_index("c")
        s = lax.axis_index("s")
        t = c * NS + s
        pltpu.sync_copy(zero_hbm.at[pl.ds(s * RPT, RPT)],
                        accum.at[pl.ds(s * RPT, RPT)])
        pltpu.sync_copy(row_hbm.at[t], rowv)
        pltpu.sync_copy(col_hbm.at[t], colv)
        plsc.subcore_barrier()
        nch = jnp.where(c == 0, CH0, CH1)

        @pl.loop(0, nch)
        def _(j):
            pltpu.sync_copy(hp_hbm.at[rowv.at[j]], buf)
            pltpu.sync_copy(buf, accum.at[colv.at[j]], add=True)

        plsc.subcore_barrier()
        pltpu.sync_copy(accum.at[pl.ds(s * RPT, RPT)],
                        out_hbm.at[c, pl.ds(s * RPT, RPT)])

    return _k


_spmm = _make_spmm(DH)
_spmm32 = _make_spmm(32)


@functools.partial(
    pl.kernel,
    out_type=jax.ShapeDtypeStruct((NC, NP, 16), jnp.float32),
    mesh=_vmesh,
    scratch_types=[
        pltpu.VMEM((CHP, K), jnp.int32),
        pltpu.VMEM((K, 16), jnp.float32),
        pltpu.VMEM_SHARED((NP, 16), jnp.float32),
    ],
    compiler_params=_sc_params,
)
def _deg(col_hbm, zero_hbm, out_hbm, colv, onesb, dacc):
    c = lax.axis_index("c")
    s = lax.axis_index("s")
    t = c * NS + s
    pltpu.sync_copy(zero_hbm.at[pl.ds(s * RPT, RPT)], dacc.at[pl.ds(s * RPT, RPT)])
    pltpu.sync_copy(col_hbm.at[t], colv)

    @pl.loop(0, K)
    def _(i):
        onesb[i, :] = jnp.ones((16,), jnp.float32)

    plsc.subcore_barrier()
    nch = jnp.where(c == 0, CH0, CH1)

    @pl.loop(0, nch)
    def _(j):
        pltpu.sync_copy(onesb, dacc.at[colv.at[j]], add=True)

    plsc.subcore_barrier()
    pltpu.sync_copy(dacc.at[pl.ds(s * RPT, RPT)],
                    out_hbm.at[c, pl.ds(s * RPT, RPT)])


def _mm_body(x_ref, w_ref, o_ref):
    o_ref[...] = jnp.dot(x_ref[...], w_ref[...], preferred_element_type=jnp.float32)


_mm1 = pl.pallas_call(
    _mm_body,
    grid=(G,),
    in_specs=[pl.BlockSpec((BLK, DI), lambda i: (i, 0)),
              pl.BlockSpec((DI, DH), lambda i: (0, 0))],
    out_specs=pl.BlockSpec((BLK, DH), lambda i: (i, 0)),
    out_shape=jax.ShapeDtypeStruct((NP, DH), jnp.float32),
)


def _dishp_body(dp_ref, xw_ref, dis_ref, hp_ref):
    deg = dp_ref[0, :, 0:1] + dp_ref[1, :, 0:1] + 1.0
    dis = lax.rsqrt(deg)
    dis_ref[...] = dis
    hp_ref[...] = dis * xw_ref[...]


_dishp = pl.pallas_call(
    _dishp_body,
    out_shape=(jax.ShapeDtypeStruct((NP, 1), jnp.float32),
               jax.ShapeDtypeStruct((NP, DH), jnp.float32)),
)


def _layer_body(acc_ref, hp_ref, dis_ref, b_ref, w_ref, o_ref):
    pre = dis_ref[...] * (acc_ref[0] + acc_ref[1] + hp_ref[...]) + b_ref[...]
    h = jnp.maximum(pre, 0.0)
    o_ref[...] = dis_ref[...] * jnp.dot(h, w_ref[...], preferred_element_type=jnp.float32)


_layer = pl.pallas_call(
    _layer_body,
    grid=(G,),
    in_specs=[pl.BlockSpec((NC, BLK, DH), lambda i: (0, i, 0)),
              pl.BlockSpec((BLK, DH), lambda i: (i, 0)),
              pl.BlockSpec((BLK, 1), lambda i: (i, 0)),
              pl.BlockSpec((1, DH), lambda i: (0, 0)),
              pl.BlockSpec((DH, DH), lambda i: (0, 0))],
    out_specs=pl.BlockSpec((BLK, DH), lambda i: (i, 0)),
    out_shape=jax.ShapeDtypeStruct((NP, DH), jnp.float32),
)


_layer5 = pl.pallas_call(
    _layer_body,
    grid=(G,),
    in_specs=[pl.BlockSpec((NC, BLK, DH), lambda i: (0, i, 0)),
              pl.BlockSpec((BLK, DH), lambda i: (i, 0)),
              pl.BlockSpec((BLK, 1), lambda i: (i, 0)),
              pl.BlockSpec((1, DH), lambda i: (0, 0)),
              pl.BlockSpec((DH, 32), lambda i: (0, 0))],
    out_specs=pl.BlockSpec((BLK, 32), lambda i: (i, 0)),
    out_shape=jax.ShapeDtypeStruct((NP, 32), jnp.float32),
)


def _head1_body(acc_ref, hp_ref, dis_ref, b5_ref, batch_ref,
                h5_ref, sums_ref, cnt_ref):
    i = pl.program_id(0)
    outsp = dis_ref[...] * (acc_ref[0] + acc_ref[1] + hp_ref[...])
    h5 = outsp[:, :DL] + b5_ref[...]
    h5_ref[...] = h5
    onehot = (batch_ref[...] == lax.broadcasted_iota(jnp.int32, (1, B), 1)
              ).astype(jnp.float32)
    ps = lax.dot_general(onehot, h5, (((0,), (0,)), ((), ())),
                         preferred_element_type=jnp.float32, precision=lax.Precision.HIGHEST)
    pc = lax.dot_general(onehot, jnp.ones((BLK, 1), jnp.float32),
                         (((0,), (0,)), ((), ())),
                         preferred_element_type=jnp.float32, precision=lax.Precision.HIGHEST)

    @pl.when(i == 0)
    def _():
        sums_ref[...] = jnp.zeros_like(sums_ref)
        cnt_ref[...] = jnp.zeros_like(cnt_ref)

    sums_ref[...] += ps
    cnt_ref[...] += pc


_head1 = pl.pallas_call(
    _head1_body,
    grid=(G,),
    in_specs=[pl.BlockSpec((NC, BLK, 32), lambda i: (0, i, 0)),
              pl.BlockSpec((BLK, 32), lambda i: (i, 0)),
              pl.BlockSpec((BLK, 1), lambda i: (i, 0)),
              pl.BlockSpec((1, DL), lambda i: (0, 0)),
              pl.BlockSpec((BLK, 1), lambda i: (i, 0))],
    out_specs=(pl.BlockSpec((BLK, DL), lambda i: (i, 0)),
               pl.BlockSpec((B, DL), lambda i: (0, 0)),
               pl.BlockSpec((B, 1), lambda i: (0, 0))),
    out_shape=(jax.ShapeDtypeStruct((NP, DL), jnp.float32),
               jax.ShapeDtypeStruct((B, DL), jnp.float32),
               jax.ShapeDtypeStruct((B, 1), jnp.float32)),
)


def _head2_body(h5_ref, sums_ref, cntv_ref, cnts_ref, base_ref,
                wl1_ref, bl1_ref, wl2_ref, bl2_ref, o_ref):
    rows = []
    off = jnp.int32(0)
    for b in range(B):
        pieces = []
        for l in range(LQ):
            bs = base_ref[b, l]
            idxv = jnp.clip(off + bs, 0, N - 1)
            rv = h5_ref[pl.ds(idxv, 1), :]
            rv = jnp.where(bs != 0, rv, 0.0)
            pieces.append(rv)
        rows.append(jnp.concatenate(pieces, axis=1))
        off = off + cnts_ref[b, 0].astype(jnp.int32)
    xs = jnp.concatenate(rows, axis=0)
    xg = sums_ref[...] / jnp.maximum(cntv_ref[...], 1.0)
    z = jnp.concatenate([xs, xg], axis=1)
    z = jnp.maximum(
        jnp.dot(z, wl1_ref[...], preferred_element_type=jnp.float32) + bl1_ref[...],
        0.0)
    o_ref[...] = jnp.dot(z, wl2_ref[...], preferred_element_type=jnp.float32) + bl2_ref[...]


_head2 = pl.pallas_call(
    _head2_body,
    in_specs=[pl.BlockSpec(memory_space=pltpu.VMEM),
              pl.BlockSpec(memory_space=pltpu.VMEM),
              pl.BlockSpec(memory_space=pltpu.VMEM),
              pl.BlockSpec(memory_space=pltpu.SMEM),
              pl.BlockSpec(memory_space=pltpu.SMEM),
              pl.BlockSpec(memory_space=pltpu.VMEM),
              pl.BlockSpec(memory_space=pltpu.VMEM),
              pl.BlockSpec(memory_space=pltpu.VMEM),
              pl.BlockSpec(memory_space=pltpu.VMEM)],
    out_shape=jax.ShapeDtypeStruct((B, 1), jnp.float32),
)


def kernel(x, edge_index, batch, base, W1, b1, W2, b2, W3, b3, W4, b4,
           W5, b5, Wl1, bl1, Wl2, bl2):
    def _pack(a):
        e0 = NS * CH0 * K
        c0 = a[:e0].reshape(NS, CH0, K)
        pad1 = jnp.full((NS * CH1 * K - (E - e0),), N, jnp.int32)
        c1 = jnp.concatenate([a[e0:], pad1]).reshape(NS, CH1, K)
        out = jnp.full((NW, CHP, K), N, jnp.int32)
        out = out.at[:NS, :CH0].set(c0)
        out = out.at[NS:, :CH1].set(c1)
        return out

    rowp = _pack(edge_index[0])
    colp = _pack(edge_index[1])
    xp = jnp.pad(x.astype(jnp.float32), ((0, NP - N), (0, 0)))
    batchp = jnp.pad(batch, (0, NP - N), constant_values=-1)[:, None]
    z64 = jnp.zeros((NP, DH), jnp.float32)
    z32 = jnp.zeros((NP, 32), jnp.float32)
    z16 = jnp.zeros((NP, 16), jnp.float32)
    W5p = jnp.pad(W5, ((0, 0), (0, 32 - DL)))

    xw = _mm1(xp, W1)
    degparts = _deg(colp, z16)
    dis, hp = _dishp(degparts, xw)

    for bb, Wn in ((b1, W2), (b2, W3), (b3, W4)):
        acc = _spmm(hp, rowp, colp, z64)
        hp = _layer(acc, hp, dis, bb[None, :], Wn)
    acc = _spmm(hp, rowp, colp, z64)
    hp = _layer5(acc, hp, dis, b4[None, :], W5p)
    acc = _spmm32(hp, rowp, colp, z32)
    h5, sums, cnt = _head1(acc, hp, dis, b5[None, :], batchp)
    out = _head2(h5, sums, cnt, cnt, base, Wl1, bl1[None, :], Wl2,
                 bl2[None, :])
    return out


# speedup vs baseline: 1.5313x; 1.5313x over previous
import functools

import jax
import jax.numpy as jnp
from jax import lax
from jax.experimental import pallas as pl
from jax.experimental.pallas import tpu as pltpu
from jax.experimental.pallas import tpu_sc as plsc

N = 10000
E = 320000
DI = 128
DH = 64
DL = 20
B = 8
LQ = 10

NC = 2
NS = 16
NW = NC * NS
K = 256
CH0 = 26
CH1 = 53
CHP = max(CH0, CH1)
NP = 10240
RPT = NP // NS
BLK = 1024
G = NP // BLK

_vmesh = plsc.VectorSubcoreMesh(core_axis_name="c", subcore_axis_name="s")
_sc_params = pltpu.CompilerParams(use_tc_tiling_on_sc=False)


def _make_spmm(DW):
    @functools.partial(
        pl.kernel,
        out_type=jax.ShapeDtypeStruct((NC, NP, DW), jnp.float32),
        mesh=_vmesh,
        scratch_types=[
            pltpu.VMEM((CHP, K), jnp.int32),
            pltpu.VMEM((CHP, K), jnp.int32),
            pltpu.VMEM((K, DW), jnp.float32),
            pltpu.VMEM_SHARED((NP, DW), jnp.float32),
        ],
        compiler_params=_sc_params,
    )
    def _k(hp_hbm, row_hbm, col_hbm, zero_hbm, out_hbm, rowv, colv, buf,
           accum):
        c = lax.axis_index("c")
        s = lax.axis_index("s")
        t = c * NS + s
        pltpu.sync_copy(zero_hbm.at[pl.ds(s * RPT, RPT)],
                        accum.at[pl.ds(s * RPT, RPT)])
        pltpu.sync_copy(row_hbm.at[t], rowv)
        pltpu.sync_copy(col_hbm.at[t], colv)
        plsc.subcore_barrier()
        nch = jnp.where(c == 0, CH0, CH1)

        @pl.loop(0, nch)
        def _(j):
            pltpu.sync_copy(hp_hbm.at[rowv.at[j]], buf)
            pltpu.sync_copy(buf, accum.at[colv.at[j]], add=True)

        plsc.subcore_barrier()
        pltpu.sync_copy(accum.at[pl.ds(s * RPT, RPT)],
                        out_hbm.at[c, pl.ds(s * RPT, RPT)])

    return _k


_spmm = _make_spmm(DH)
_spmm32 = _make_spmm(32)


@functools.partial(
    pl.kernel,
    out_type=jax.ShapeDtypeStruct((NC, NP, 16), jnp.float32),
    mesh=_vmesh,
    scratch_types=[
        pltpu.VMEM((CHP, K), jnp.int32),
        pltpu.VMEM((K, 16), jnp.float32),
        pltpu.VMEM_SHARED((NP, 16), jnp.float32),
    ],
    compiler_params=_sc_params,
)
def _deg(col_hbm, zero_hbm, out_hbm, colv, onesb, dacc):
    c = lax.axis_index("c")
    s = lax.axis_index("s")
    t = c * NS + s
    pltpu.sync_copy(zero_hbm.at[pl.ds(s * RPT, RPT)], dacc.at[pl.ds(s * RPT, RPT)])
    pltpu.sync_copy(col_hbm.at[t], colv)

    @pl.loop(0, K)
    def _(i):
        onesb[i, :] = jnp.ones((16,), jnp.float32)

    plsc.subcore_barrier()
    nch = jnp.where(c == 0, CH0, CH1)

    @pl.loop(0, nch)
    def _(j):
        pltpu.sync_copy(onesb, dacc.at[colv.at[j]], add=True)

    plsc.subcore_barrier()
    pltpu.sync_copy(dacc.at[pl.ds(s * RPT, RPT)],
                    out_hbm.at[c, pl.ds(s * RPT, RPT)])


def _mm_body(x_ref, w_ref, o_ref):
    o_ref[...] = jnp.dot(x_ref[...], w_ref[...], preferred_element_type=jnp.float32)


_mm1 = pl.pallas_call(
    _mm_body,
    grid=(G,),
    in_specs=[pl.BlockSpec((BLK, DI), lambda i: (i, 0)),
              pl.BlockSpec((DI, DH), lambda i: (0, 0))],
    out_specs=pl.BlockSpec((BLK, DH), lambda i: (i, 0)),
    out_shape=jax.ShapeDtypeStruct((NP, DH), jnp.float32),
)


def _dishp_body(dp_ref, xw_ref, dis_ref, hp_ref):
    deg = dp_ref[0, :, 0:1] + dp_ref[1, :, 0:1] + 1.0
    dis = lax.rsqrt(deg)
    dis_ref[...] = dis
    hp_ref[...] = dis * xw_ref[...]


_dishp = pl.pallas_call(
    _dishp_body,
    out_shape=(jax.ShapeDtypeStruct((NP, 1), jnp.float32),
               jax.ShapeDtypeStruct((NP, DH), jnp.float32)),
)


def _layer_body(acc_ref, hp_ref, dis_ref, b_ref, w_ref, o_ref):
    pre = dis_ref[...] * (acc_ref[0] + acc_ref[1] + hp_ref[...]) + b_ref[...]
    h = jnp.maximum(pre, 0.0)
    o_ref[...] = dis_ref[...] * jnp.dot(h, w_ref[...], preferred_element_type=jnp.float32)


_layer = pl.pallas_call(
    _layer_body,
    grid=(G,),
    in_specs=[pl.BlockSpec((NC, BLK, DH), lambda i: (0, i, 0)),
              pl.BlockSpec((BLK, DH), lambda i: (i, 0)),
              pl.BlockSpec((BLK, 1), lambda i: (i, 0)),
              pl.BlockSpec((1, DH), lambda i: (0, 0)),
              pl.BlockSpec((DH, DH), lambda i: (0, 0))],
    out_specs=pl.BlockSpec((BLK, DH), lambda i: (i, 0)),
    out_shape=jax.ShapeDtypeStruct((NP, DH), jnp.float32),
)


_layer5 = pl.pallas_call(
    _layer_body,
    grid=(G,),
    in_specs=[pl.BlockSpec((NC, BLK, DH), lambda i: (0, i, 0)),
              pl.BlockSpec((BLK, DH), lambda i: (i, 0)),
              pl.BlockSpec((BLK, 1), lambda i: (i, 0)),
              pl.BlockSpec((1, DH), lambda i: (0, 0)),
              pl.BlockSpec((DH, 32), lambda i: (0, 0))],
    out_specs=pl.BlockSpec((BLK, 32), lambda i: (i, 0)),
    out_shape=jax.ShapeDtypeStruct((NP, 32), jnp.float32),
)


def _head1_body(acc_ref, hp_ref, dis_ref, b5_ref, batch_ref,
                h5_ref, sums_ref, cnt_ref):
    i = pl.program_id(0)
    outsp = dis_ref[...] * (acc_ref[0] + acc_ref[1] + hp_ref[...])
    h5 = outsp[:, :DL] + b5_ref[...]
    h5_ref[...] = h5
    onehot = (batch_ref[...] == lax.broadcasted_iota(jnp.int32, (1, B), 1)
              ).astype(jnp.float32)
    ps = lax.dot_general(onehot, h5, (((0,), (0,)), ((), ())),
                         preferred_element_type=jnp.float32, precision=lax.Precision.HIGHEST)
    pc = lax.dot_general(onehot, jnp.ones((BLK, 1), jnp.float32),
                         (((0,), (0,)), ((), ())),
                         preferred_element_type=jnp.float32, precision=lax.Precision.HIGHEST)

    @pl.when(i == 0)
    def _():
        sums_ref[...] = jnp.zeros_like(sums_ref)
        cnt_ref[...] = jnp.zeros_like(cnt_ref)

    sums_ref[...] += ps
    cnt_ref[...] += pc


_head1 = pl.pallas_call(
    _head1_body,
    grid=(G,),
    in_specs=[pl.BlockSpec((NC, BLK, 32), lambda i: (0, i, 0)),
              pl.BlockSpec((BLK, 32), lambda i: (i, 0)),
              pl.BlockSpec((BLK, 1), lambda i: (i, 0)),
              pl.BlockSpec((1, DL), lambda i: (0, 0)),
              pl.BlockSpec((BLK, 1), lambda i: (i, 0))],
    out_specs=(pl.BlockSpec((BLK, DL), lambda i: (i, 0)),
               pl.BlockSpec((B, DL), lambda i: (0, 0)),
               pl.BlockSpec((B, 1), lambda i: (0, 0))),
    out_shape=(jax.ShapeDtypeStruct((NP, DL), jnp.float32),
               jax.ShapeDtypeStruct((B, DL), jnp.float32),
               jax.ShapeDtypeStruct((B, 1), jnp.float32)),
)


def _head2_body(h5_ref, sums_ref, cntv_ref, cnts_ref, base_ref,
                wl1_ref, bl1_ref, wl2_ref, bl2_ref, o_ref):
    rows = []
    off = jnp.int32(0)
    for b in range(B):
        pieces = []
        for l in range(LQ):
            bs = base_ref[b, l]
            idxv = jnp.clip(off + bs, 0, N - 1)
            rv = h5_ref[pl.ds(idxv, 1), :]
            rv = jnp.where(bs != 0, rv, 0.0)
            pieces.append(rv)
        rows.append(jnp.concatenate(pieces, axis=1))
        off = off + cnts_ref[b, 0].astype(jnp.int32)
    xs = jnp.concatenate(rows, axis=0)
    xg = sums_ref[...] / jnp.maximum(cntv_ref[...], 1.0)
    z = jnp.concatenate([xs, xg], axis=1)
    z = jnp.maximum(
        jnp.dot(z, wl1_ref[...], preferred_element_type=jnp.float32) + bl1_ref[...],
        0.0)
    o_ref[...] = jnp.dot(z, wl2_ref[...], preferred_element_type=jnp.float32) + bl2_ref[...]


_head2 = pl.pallas_call(
    _head2_body,
    in_specs=[pl.BlockSpec(memory_space=pltpu.VMEM),
              pl.BlockSpec(memory_space=pltpu.VMEM),
              pl.BlockSpec(memory_space=pltpu.VMEM),
              pl.BlockSpec(memory_space=pltpu.SMEM),
              pl.BlockSpec(memory_space=pltpu.SMEM),
              pl.BlockSpec(memory_space=pltpu.VMEM),
              pl.BlockSpec(memory_space=pltpu.VMEM),
              pl.BlockSpec(memory_space=pltpu.VMEM),
              pl.BlockSpec(memory_space=pltpu.VMEM)],
    out_shape=jax.ShapeDtypeStruct((B, 1), jnp.float32),
)


def kernel(x, edge_index, batch, base, W1, b1, W2, b2, W3, b3, W4, b4,
           W5, b5, Wl1, bl1, Wl2, bl2):
    def _pack(a):
        e0 = NS * CH0 * K
        c0 = a[:e0].reshape(NS, CH0, K)
        pad1 = jnp.full((NS * CH1 * K - (E - e0),), N, jnp.int32)
        c1 = jnp.concatenate([a[e0:], pad1]).reshape(NS, CH1, K)
        out = jnp.full((NW, CHP, K), N, jnp.int32)
        out = out.at[:NS, :CH0].set(c0)
        out = out.at[NS:, :CH1].set(c1)
        return out

    rowp = _pack(edge_index[0])
    colp = _pack(edge_index[1])
    xp = jnp.pad(x.astype(jnp.float32), ((0, NP - N), (0, 0)))
    batchp = jnp.pad(batch, (0, NP - N), constant_values=-1)[:, None]
    z64 = jnp.zeros((NP, DH), jnp.float32)
    z32 = jnp.zeros((NP, 32), jnp.float32)
    z16 = jnp.zeros((NP, 16), jnp.float32)
    W5p = jnp.pad(W5, ((0, 0), (0, 32 - DL)))

    xw = _mm1(xp, W1)
    degparts = _deg(colp, z16)
    dis, hp = _dishp(degparts, xw)

    for bb, Wn in ((b1, W2), (b2, W3), (b3, W4)):
        acc = _spmm(hp, rowp, colp, z64)
        hp = _layer(acc, hp, dis, bb[None, :], Wn)
    acc = _spmm(hp, rowp, colp, z64)
    hp = _layer5(acc, hp, dis, b4[None, :], W5p)
    acc = _spmm32(hp, rowp, colp, z32)
    h5, sums, cnt = _head1(acc, hp, dis, b5[None, :], batchp)
    out = _head2(h5, sums, cnt, cnt, base, Wl1, bl1[None, :], Wl2,
                 bl2[None, :])
    return out


# speedup vs baseline: 1.5479x; 1.0108x over previous
import functools

import jax
import jax.numpy as jnp
from jax import lax
from jax.experimental import pallas as pl
from jax.experimental.pallas import tpu as pltpu
from jax.experimental.pallas import tpu_sc as plsc

N = 10000
E = 320000
DI = 128
DH = 64
DL = 20
B = 8
LQ = 10

NC = 2
NS = 16
NW = NC * NS
K = 128
CH0 = 52
CH1 = 105
CHP = max(CH0, CH1)
NP = 10240
RPT = NP // NS
BLK = 1024
G = NP // BLK

_vmesh = plsc.VectorSubcoreMesh(core_axis_name="c", subcore_axis_name="s")
_sc_params = pltpu.CompilerParams(use_tc_tiling_on_sc=False)


def _make_spmm(DW):
    @functools.partial(
        pl.kernel,
        out_type=jax.ShapeDtypeStruct((NC, NP, DW), jnp.float32),
        mesh=_vmesh,
        scratch_types=[
            pltpu.VMEM((CHP, K), jnp.int32),
            pltpu.VMEM((CHP, K), jnp.int32),
            pltpu.VMEM((K, DW), jnp.float32),
            pltpu.VMEM_SHARED((NP, DW), jnp.float32),
        ],
        compiler_params=_sc_params,
    )
    def _k(hp_hbm, row_hbm, col_hbm, zero_hbm, out_hbm, rowv, colv, buf,
           accum):
        c = lax.axis_index("c")
        s = lax.axis_index("s")
        t = c * NS + s
        pltpu.sync_copy(zero_hbm.at[pl.ds(s * RPT, RPT)],
                        accum.at[pl.ds(s * RPT, RPT)])
        pltpu.sync_copy(row_hbm.at[t], rowv)
        pltpu.sync_copy(col_hbm.at[t], colv)
        plsc.subcore_barrier()
        nch = jnp.where(c == 0, CH0, CH1)

        @pl.loop(0, nch)
        def _(j):
            pltpu.sync_copy(hp_hbm.at[rowv.at[j]], buf)
            pltpu.sync_copy(buf, accum.at[colv.at[j]], add=True)

        plsc.subcore_barrier()
        pltpu.sync_copy(accum.at[pl.ds(s * RPT, RPT)],
                        out_hbm.at[c, pl.ds(s * RPT, RPT)])

    return _k


_spmm = _make_spmm(DH)
_spmm32 = _make_spmm(32)


@functools.partial(
    pl.kernel,
    out_type=jax.ShapeDtypeStruct((NC, NP, 16), jnp.float32),
    mesh=_vmesh,
    scratch_types=[
        pltpu.VMEM((CHP, K), jnp.int32),
        pltpu.VMEM((K, 16), jnp.float32),
        pltpu.VMEM_SHARED((NP, 16), jnp.float32),
    ],
    compiler_params=_sc_params,
)
def _deg(col_hbm, zero_hbm, out_hbm, colv, onesb, dacc):
    c = lax.axis_index("c")
    s = lax.axis_index("s")
    t = c * NS + s
    pltpu.sync_copy(zero_hbm.at[pl.ds(s * RPT, RPT)], dacc.at[pl.ds(s * RPT, RPT)])
    pltpu.sync_copy(col_hbm.at[t], colv)

    @pl.loop(0, K)
    def _(i):
        onesb[i, :] = jnp.ones((16,), jnp.float32)

    plsc.subcore_barrier()
    nch = jnp.where(c == 0, CH0, CH1)

    @pl.loop(0, nch)
    def _(j):
        pltpu.sync_copy(onesb, dacc.at[colv.at[j]], add=True)

    plsc.subcore_barrier()
    pltpu.sync_copy(dacc.at[pl.ds(s * RPT, RPT)],
                    out_hbm.at[c, pl.ds(s * RPT, RPT)])


def _mm_body(x_ref, w_ref, o_ref):
    o_ref[...] = jnp.dot(x_ref[...], w_ref[...], preferred_element_type=jnp.float32)


_mm1 = pl.pallas_call(
    _mm_body,
    grid=(G,),
    in_specs=[pl.BlockSpec((BLK, DI), lambda i: (i, 0)),
              pl.BlockSpec((DI, DH), lambda i: (0, 0))],
    out_specs=pl.BlockSpec((BLK, DH), lambda i: (i, 0)),
    out_shape=jax.ShapeDtypeStruct((NP, DH), jnp.float32),
)


def _dishp_body(dp_ref, xw_ref, dis_ref, hp_ref):
    deg = dp_ref[0, :, 0:1] + dp_ref[1, :, 0:1] + 1.0
    dis = lax.rsqrt(deg)
    dis_ref[...] = dis
    hp_ref[...] = dis * xw_ref[...]


_dishp = pl.pallas_call(
    _dishp_body,
    out_shape=(jax.ShapeDtypeStruct((NP, 1), jnp.float32),
               jax.ShapeDtypeStruct((NP, DH), jnp.float32)),
)


def _layer_body(acc_ref, hp_ref, dis_ref, b_ref, w_ref, o_ref):
    pre = dis_ref[...] * (acc_ref[0] + acc_ref[1] + hp_ref[...]) + b_ref[...]
    h = jnp.maximum(pre, 0.0)
    o_ref[...] = dis_ref[...] * jnp.dot(h, w_ref[...], preferred_element_type=jnp.float32)


_layer = pl.pallas_call(
    _layer_body,
    grid=(G,),
    in_specs=[pl.BlockSpec((NC, BLK, DH), lambda i: (0, i, 0)),
              pl.BlockSpec((BLK, DH), lambda i: (i, 0)),
              pl.BlockSpec((BLK, 1), lambda i: (i, 0)),
              pl.BlockSpec((1, DH), lambda i: (0, 0)),
              pl.BlockSpec((DH, DH), lambda i: (0, 0))],
    out_specs=pl.BlockSpec((BLK, DH), lambda i: (i, 0)),
    out_shape=jax.ShapeDtypeStruct((NP, DH), jnp.float32),
)


_layer5 = pl.pallas_call(
    _layer_body,
    grid=(G,),
    in_specs=[pl.BlockSpec((NC, BLK, DH), lambda i: (0, i, 0)),
              pl.BlockSpec((BLK, DH), lambda i: (i, 0)),
              pl.BlockSpec((BLK, 1), lambda i: (i, 0)),
              pl.BlockSpec((1, DH), lambda i: (0, 0)),
              pl.BlockSpec((DH, 32), lambda i: (0, 0))],
    out_specs=pl.BlockSpec((BLK, 32), lambda i: (i, 0)),
    out_shape=jax.ShapeDtypeStruct((NP, 32), jnp.float32),
)


def _head1_body(acc_ref, hp_ref, dis_ref, b5_ref, batch_ref,
                h5_ref, sums_ref, cnt_ref):
    i = pl.program_id(0)
    outsp = dis_ref[...] * (acc_ref[0] + acc_ref[1] + hp_ref[...])
    h5 = outsp[:, :DL] + b5_ref[...]
    h5_ref[...] = h5
    onehot = (batch_ref[...] == lax.broadcasted_iota(jnp.int32, (1, B), 1)
              ).astype(jnp.float32)
    ps = lax.dot_general(onehot, h5, (((0,), (0,)), ((), ())),
                         preferred_element_type=jnp.float32, precision=lax.Precision.HIGHEST)
    pc = lax.dot_general(onehot, jnp.ones((BLK, 1), jnp.float32),
                         (((0,), (0,)), ((), ())),
                         preferred_element_type=jnp.float32, precision=lax.Precision.HIGHEST)

    @pl.when(i == 0)
    def _():
        sums_ref[...] = jnp.zeros_like(sums_ref)
        cnt_ref[...] = jnp.zeros_like(cnt_ref)

    sums_ref[...] += ps
    cnt_ref[...] += pc


_head1 = pl.pallas_call(
    _head1_body,
    grid=(G,),
    in_specs=[pl.BlockSpec((NC, BLK, 32), lambda i: (0, i, 0)),
              pl.BlockSpec((BLK, 32), lambda i: (i, 0)),
              pl.BlockSpec((BLK, 1), lambda i: (i, 0)),
              pl.BlockSpec((1, DL), lambda i: (0, 0)),
              pl.BlockSpec((BLK, 1), lambda i: (i, 0))],
    out_specs=(pl.BlockSpec((BLK, DL), lambda i: (i, 0)),
               pl.BlockSpec((B, DL), lambda i: (0, 0)),
               pl.BlockSpec((B, 1), lambda i: (0, 0))),
    out_shape=(jax.ShapeDtypeStruct((NP, DL), jnp.float32),
               jax.ShapeDtypeStruct((B, DL), jnp.float32),
               jax.ShapeDtypeStruct((B, 1), jnp.float32)),
)


def _head2_body(h5_ref, sums_ref, cntv_ref, cnts_ref, base_ref,
                wl1_ref, bl1_ref, wl2_ref, bl2_ref, o_ref):
    rows = []
    off = jnp.int32(0)
    for b in range(B):
        pieces = []
        for l in range(LQ):
            bs = base_ref[b, l]
            idxv = jnp.clip(off + bs, 0, N - 1)
            rv = h5_ref[pl.ds(idxv, 1), :]
            rv = jnp.where(bs != 0, rv, 0.0)
            pieces.append(rv)
        rows.append(jnp.concatenate(pieces, axis=1))
        off = off + cnts_ref[b, 0].astype(jnp.int32)
    xs = jnp.concatenate(rows, axis=0)
    xg = sums_ref[...] / jnp.maximum(cntv_ref[...], 1.0)
    z = jnp.concatenate([xs, xg], axis=1)
    z = jnp.maximum(
        jnp.dot(z, wl1_ref[...], preferred_element_type=jnp.float32) + bl1_ref[...],
        0.0)
    o_ref[...] = jnp.dot(z, wl2_ref[...], preferred_element_type=jnp.float32) + bl2_ref[...]


_head2 = pl.pallas_call(
    _head2_body,
    in_specs=[pl.BlockSpec(memory_space=pltpu.VMEM),
              pl.BlockSpec(memory_space=pltpu.VMEM),
              pl.BlockSpec(memory_space=pltpu.VMEM),
              pl.BlockSpec(memory_space=pltpu.SMEM),
              pl.BlockSpec(memory_space=pltpu.SMEM),
              pl.BlockSpec(memory_space=pltpu.VMEM),
              pl.BlockSpec(memory_space=pltpu.VMEM),
              pl.BlockSpec(memory_space=pltpu.VMEM),
              pl.BlockSpec(memory_space=pltpu.VMEM)],
    out_shape=jax.ShapeDtypeStruct((B, 1), jnp.float32),
)


def kernel(x, edge_index, batch, base, W1, b1, W2, b2, W3, b3, W4, b4,
           W5, b5, Wl1, bl1, Wl2, bl2):
    def _pack(a):
        e0 = NS * CH0 * K
        c0 = a[:e0].reshape(NS, CH0, K)
        pad1 = jnp.full((NS * CH1 * K - (E - e0),), N, jnp.int32)
        c1 = jnp.concatenate([a[e0:], pad1]).reshape(NS, CH1, K)
        out = jnp.full((NW, CHP, K), N, jnp.int32)
        out = out.at[:NS, :CH0].set(c0)
        out = out.at[NS:, :CH1].set(c1)
        return out

    rowp = _pack(edge_index[0])
    colp = _pack(edge_index[1])
    xp = jnp.pad(x.astype(jnp.float32), ((0, NP - N), (0, 0)))
    batchp = jnp.pad(batch, (0, NP - N), constant_values=-1)[:, None]
    z64 = jnp.zeros((NP, DH), jnp.float32)
    z32 = jnp.zeros((NP, 32), jnp.float32)
    z16 = jnp.zeros((NP, 16), jnp.float32)
    W5p = jnp.pad(W5, ((0, 0), (0, 32 - DL)))

    xw = _mm1(xp, W1)
    degparts = _deg(colp, z16)
    dis, hp = _dishp(degparts, xw)

    for bb, Wn in ((b1, W2), (b2, W3), (b3, W4)):
        acc = _spmm(hp, rowp, colp, z64)
        hp = _layer(acc, hp, dis, bb[None, :], Wn)
    acc = _spmm(hp, rowp, colp, z64)
    hp = _layer5(acc, hp, dis, b4[None, :], W5p)
    acc = _spmm32(hp, rowp, colp, z32)
    h5, sums, cnt = _head1(acc, hp, dis, b5[None, :], batchp)
    out = _head2(h5, sums, cnt, cnt, base, Wl1, bl1[None, :], Wl2,
                 bl2[None, :])
    return out


# speedup vs baseline: 1.9888x; 1.2848x over previous
import functools

import jax
import jax.numpy as jnp
from jax import lax
from jax.experimental import pallas as pl
from jax.experimental.pallas import tpu as pltpu
from jax.experimental.pallas import tpu_sc as plsc

N = 10000
E = 320000
DI = 128
DH = 64
DL = 20
B = 8
LQ = 10

NC = 2
NS = 16
NW = NC * NS
K = 128
CH0 = 52
CH1 = 105
CHP = max(CH0, CH1)
NP = 10240
RPT = NP // NS
BLK = 1024
G = NP // BLK

_vmesh = plsc.VectorSubcoreMesh(core_axis_name="c", subcore_axis_name="s")
_sc_params = pltpu.CompilerParams(use_tc_tiling_on_sc=False)


def _make_spmm(DW):
    @functools.partial(
        pl.kernel,
        out_type=jax.ShapeDtypeStruct((NC, NP, DW), jnp.float32),
        mesh=_vmesh,
        scratch_types=[
            pltpu.VMEM((CHP, K), jnp.int32),
            pltpu.VMEM((CHP, K), jnp.int32),
            pltpu.VMEM((K, DW), jnp.float32),
            pltpu.VMEM_SHARED((NP, DW), jnp.float32),
            pltpu.VMEM_SHARED((NP, DW), jnp.float32),
        ],
        compiler_params=_sc_params,
    )
    def _k(hp_hbm, row_hbm, col_hbm, zero_hbm, out_hbm, rowv, colv, buf,
           accum, hps):
        c = lax.axis_index("c")
        s = lax.axis_index("s")
        t = c * NS + s
        pltpu.sync_copy(zero_hbm.at[pl.ds(s * RPT, RPT)],
                        accum.at[pl.ds(s * RPT, RPT)])
        pltpu.sync_copy(hp_hbm.at[pl.ds(s * RPT, RPT)],
                        hps.at[pl.ds(s * RPT, RPT)])
        pltpu.sync_copy(row_hbm.at[t], rowv)
        pltpu.sync_copy(col_hbm.at[t], colv)
        plsc.subcore_barrier()
        nch = jnp.where(c == 0, CH0, CH1)

        @pl.loop(0, nch)
        def _(j):
            pltpu.sync_copy(hps.at[rowv.at[j]], buf)
            pltpu.sync_copy(buf, accum.at[colv.at[j]], add=True)

        plsc.subcore_barrier()
        pltpu.sync_copy(accum.at[pl.ds(s * RPT, RPT)],
                        out_hbm.at[c, pl.ds(s * RPT, RPT)])

    return _k


_spmm = _make_spmm(DH)
_spmm32 = _make_spmm(32)


@functools.partial(
    pl.kernel,
    out_type=jax.ShapeDtypeStruct((NC, NP, 16), jnp.float32),
    mesh=_vmesh,
    scratch_types=[
        pltpu.VMEM((CHP, K), jnp.int32),
        pltpu.VMEM((K, 16), jnp.float32),
        pltpu.VMEM_SHARED((NP, 16), jnp.float32),
    ],
    compiler_params=_sc_params,
)
def _deg(col_hbm, zero_hbm, out_hbm, colv, onesb, dacc):
    c = lax.axis_index("c")
    s = lax.axis_index("s")
    t = c * NS + s
    pltpu.sync_copy(zero_hbm.at[pl.ds(s * RPT, RPT)], dacc.at[pl.ds(s * RPT, RPT)])
    pltpu.sync_copy(col_hbm.at[t], colv)

    @pl.loop(0, K)
    def _(i):
        onesb[i, :] = jnp.ones((16,), jnp.float32)

    plsc.subcore_barrier()
    nch = jnp.where(c == 0, CH0, CH1)

    @pl.loop(0, nch)
    def _(j):
        pltpu.sync_copy(onesb, dacc.at[colv.at[j]], add=True)

    plsc.subcore_barrier()
    pltpu.sync_copy(dacc.at[pl.ds(s * RPT, RPT)],
                    out_hbm.at[c, pl.ds(s * RPT, RPT)])


def _mm_body(x_ref, w_ref, o_ref):
    o_ref[...] = jnp.dot(x_ref[...], w_ref[...], preferred_element_type=jnp.float32)


_mm1 = pl.pallas_call(
    _mm_body,
    grid=(G,),
    in_specs=[pl.BlockSpec((BLK, DI), lambda i: (i, 0)),
              pl.BlockSpec((DI, DH), lambda i: (0, 0))],
    out_specs=pl.BlockSpec((BLK, DH), lambda i: (i, 0)),
    out_shape=jax.ShapeDtypeStruct((NP, DH), jnp.float32),
)


def _dishp_body(dp_ref, xw_ref, dis_ref, hp_ref):
    deg = dp_ref[0, :, 0:1] + dp_ref[1, :, 0:1] + 1.0
    dis = lax.rsqrt(deg)
    dis_ref[...] = dis
    hp_ref[...] = dis * xw_ref[...]


_dishp = pl.pallas_call(
    _dishp_body,
    out_shape=(jax.ShapeDtypeStruct((NP, 1), jnp.float32),
               jax.ShapeDtypeStruct((NP, DH), jnp.float32)),
)


def _layer_body(acc_ref, hp_ref, dis_ref, b_ref, w_ref, o_ref):
    pre = dis_ref[...] * (acc_ref[0] + acc_ref[1] + hp_ref[...]) + b_ref[...]
    h = jnp.maximum(pre, 0.0)
    o_ref[...] = dis_ref[...] * jnp.dot(h, w_ref[...], preferred_element_type=jnp.float32)


_layer = pl.pallas_call(
    _layer_body,
    grid=(G,),
    in_specs=[pl.BlockSpec((NC, BLK, DH), lambda i: (0, i, 0)),
              pl.BlockSpec((BLK, DH), lambda i: (i, 0)),
              pl.BlockSpec((BLK, 1), lambda i: (i, 0)),
              pl.BlockSpec((1, DH), lambda i: (0, 0)),
              pl.BlockSpec((DH, DH), lambda i: (0, 0))],
    out_specs=pl.BlockSpec((BLK, DH), lambda i: (i, 0)),
    out_shape=jax.ShapeDtypeStruct((NP, DH), jnp.float32),
)


_layer5 = pl.pallas_call(
    _layer_body,
    grid=(G,),
    in_specs=[pl.BlockSpec((NC, BLK, DH), lambda i: (0, i, 0)),
              pl.BlockSpec((BLK, DH), lambda i: (i, 0)),
              pl.BlockSpec((BLK, 1), lambda i: (i, 0)),
              pl.BlockSpec((1, DH), lambda i: (0, 0)),
              pl.BlockSpec((DH, 32), lambda i: (0, 0))],
    out_specs=pl.BlockSpec((BLK, 32), lambda i: (i, 0)),
    out_shape=jax.ShapeDtypeStruct((NP, 32), jnp.float32),
)


def _head1_body(acc_ref, hp_ref, dis_ref, b5_ref, batch_ref,
                h5_ref, sums_ref, cnt_ref):
    i = pl.program_id(0)
    outsp = dis_ref[...] * (acc_ref[0] + acc_ref[1] + hp_ref[...])
    h5 = outsp[:, :DL] + b5_ref[...]
    h5_ref[...] = h5
    onehot = (batch_ref[...] == lax.broadcasted_iota(jnp.int32, (1, B), 1)
              ).astype(jnp.float32)
    ps = lax.dot_general(onehot, h5, (((0,), (0,)), ((), ())),
                         preferred_element_type=jnp.float32, precision=lax.Precision.HIGHEST)
    pc = lax.dot_general(onehot, jnp.ones((BLK, 1), jnp.float32),
                         (((0,), (0,)), ((), ())),
                         preferred_element_type=jnp.float32, precision=lax.Precision.HIGHEST)

    @pl.when(i == 0)
    def _():
        sums_ref[...] = jnp.zeros_like(sums_ref)
        cnt_ref[...] = jnp.zeros_like(cnt_ref)

    sums_ref[...] += ps
    cnt_ref[...] += pc


_head1 = pl.pallas_call(
    _head1_body,
    grid=(G,),
    in_specs=[pl.BlockSpec((NC, BLK, 32), lambda i: (0, i, 0)),
              pl.BlockSpec((BLK, 32), lambda i: (i, 0)),
              pl.BlockSpec((BLK, 1), lambda i: (i, 0)),
              pl.BlockSpec((1, DL), lambda i: (0, 0)),
              pl.BlockSpec((BLK, 1), lambda i: (i, 0))],
    out_specs=(pl.BlockSpec((BLK, DL), lambda i: (i, 0)),
               pl.BlockSpec((B, DL), lambda i: (0, 0)),
               pl.BlockSpec((B, 1), lambda i: (0, 0))),
    out_shape=(jax.ShapeDtypeStruct((NP, DL), jnp.float32),
               jax.ShapeDtypeStruct((B, DL), jnp.float32),
               jax.ShapeDtypeStruct((B, 1), jnp.float32)),
)


def _head2_body(h5_ref, sums_ref, cntv_ref, cnts_ref, base_ref,
                wl1_ref, bl1_ref, wl2_ref, bl2_ref, o_ref):
    rows = []
    off = jnp.int32(0)
    for b in range(B):
        pieces = []
        for l in range(LQ):
            bs = base_ref[b, l]
            idxv = jnp.clip(off + bs, 0, N - 1)
            rv = h5_ref[pl.ds(idxv, 1), :]
            rv = jnp.where(bs != 0, rv, 0.0)
            pieces.append(rv)
        rows.append(jnp.concatenate(pieces, axis=1))
        off = off + cnts_ref[b, 0].astype(jnp.int32)
    xs = jnp.concatenate(rows, axis=0)
    xg = sums_ref[...] / jnp.maximum(cntv_ref[...], 1.0)
    z = jnp.concatenate([xs, xg], axis=1)
    z = jnp.maximum(
        jnp.dot(z, wl1_ref[...], preferred_element_type=jnp.float32) + bl1_ref[...],
        0.0)
    o_ref[...] = jnp.dot(z, wl2_ref[...], preferred_element_type=jnp.float32) + bl2_ref[...]


_head2 = pl.pallas_call(
    _head2_body,
    in_specs=[pl.BlockSpec(memory_space=pltpu.VMEM),
              pl.BlockSpec(memory_space=pltpu.VMEM),
              pl.BlockSpec(memory_space=pltpu.VMEM),
              pl.BlockSpec(memory_space=pltpu.SMEM),
              pl.BlockSpec(memory_space=pltpu.SMEM),
              pl.BlockSpec(memory_space=pltpu.VMEM),
              pl.BlockSpec(memory_space=pltpu.VMEM),
              pl.BlockSpec(memory_space=pltpu.VMEM),
              pl.BlockSpec(memory_space=pltpu.VMEM)],
    out_shape=jax.ShapeDtypeStruct((B, 1), jnp.float32),
)


def kernel(x, edge_index, batch, base, W1, b1, W2, b2, W3, b3, W4, b4,
           W5, b5, Wl1, bl1, Wl2, bl2):
    def _pack(a):
        e0 = NS * CH0 * K
        c0 = a[:e0].reshape(NS, CH0, K)
        pad1 = jnp.full((NS * CH1 * K - (E - e0),), N, jnp.int32)
        c1 = jnp.concatenate([a[e0:], pad1]).reshape(NS, CH1, K)
        out = jnp.full((NW, CHP, K), N, jnp.int32)
        out = out.at[:NS, :CH0].set(c0)
        out = out.at[NS:, :CH1].set(c1)
        return out

    rowp = _pack(edge_index[0])
    colp = _pack(edge_index[1])
    xp = jnp.pad(x.astype(jnp.float32), ((0, NP - N), (0, 0)))
    batchp = jnp.pad(batch, (0, NP - N), constant_values=-1)[:, None]
    z64 = jnp.zeros((NP, DH), jnp.float32)
    z32 = jnp.zeros((NP, 32), jnp.float32)
    z16 = jnp.zeros((NP, 16), jnp.float32)
    W5p = jnp.pad(W5, ((0, 0), (0, 32 - DL)))

    xw = _mm1(xp, W1)
    degparts = _deg(colp, z16)
    dis, hp = _dishp(degparts, xw)

    for bb, Wn in ((b1, W2), (b2, W3), (b3, W4)):
        acc = _spmm(hp, rowp, colp, z64)
        hp = _layer(acc, hp, dis, bb[None, :], Wn)
    acc = _spmm(hp, rowp, colp, z64)
    hp = _layer5(acc, hp, dis, b4[None, :], W5p)
    acc = _spmm32(hp, rowp, colp, z32)
    h5, sums, cnt = _head1(acc, hp, dis, b5[None, :], batchp)
    out = _head2(h5, sums, cnt, cnt, base, Wl1, bl1[None, :], Wl2,
                 bl2[None, :])
    return out


# speedup vs baseline: 2.3671x; 1.1902x over previous
import functools

import jax
import jax.numpy as jnp
from jax import lax
from jax.experimental import pallas as pl
from jax.experimental.pallas import tpu as pltpu
from jax.experimental.pallas import tpu_sc as plsc

N = 10000
E = 320000
DI = 128
DH = 64
DL = 20
B = 8
LQ = 10

NC = 2
NS = 16
NW = NC * NS
K = 128
CH0 = 79
CH1 = 79
CHP = max(CH0, CH1)
NP = 10240
RPT = NP // NS
BLK = 1024
G = NP // BLK

_vmesh = plsc.VectorSubcoreMesh(core_axis_name="c", subcore_axis_name="s")
_sc_params = pltpu.CompilerParams(use_tc_tiling_on_sc=False)


def _make_spmm(DW):
    @functools.partial(
        pl.kernel,
        out_type=jax.ShapeDtypeStruct((NC, NP, DW), jnp.float32),
        mesh=_vmesh,
        scratch_types=[
            pltpu.VMEM((CHP, K), jnp.int32),
            pltpu.VMEM((CHP, K), jnp.int32),
            pltpu.VMEM((K, DW), jnp.float32),
            pltpu.VMEM_SHARED((NP, DW), jnp.float32),
            pltpu.VMEM_SHARED((NP, DW), jnp.float32),
        ],
        compiler_params=_sc_params,
    )
    def _k(hp_hbm, row_hbm, col_hbm, zero_hbm, out_hbm, rowv, colv, buf,
           accum, hps):
        c = lax.axis_index("c")
        s = lax.axis_index("s")
        t = c * NS + s
        pltpu.sync_copy(zero_hbm.at[pl.ds(s * RPT, RPT)],
                        accum.at[pl.ds(s * RPT, RPT)])
        pltpu.sync_copy(hp_hbm.at[pl.ds(s * RPT, RPT)],
                        hps.at[pl.ds(s * RPT, RPT)])
        pltpu.sync_copy(row_hbm.at[t], rowv)
        pltpu.sync_copy(col_hbm.at[t], colv)
        plsc.subcore_barrier()
        nch = jnp.where(c == 0, CH0, CH1)

        @pl.loop(0, nch)
        def _(j):
            pltpu.sync_copy(hps.at[rowv.at[j]], buf)
            pltpu.sync_copy(buf, accum.at[colv.at[j]], add=True)

        plsc.subcore_barrier()
        pltpu.sync_copy(accum.at[pl.ds(s * RPT, RPT)],
                        out_hbm.at[c, pl.ds(s * RPT, RPT)])

    return _k


_spmm = _make_spmm(DH)
_spmm32 = _make_spmm(32)


@functools.partial(
    pl.kernel,
    out_type=jax.ShapeDtypeStruct((NC, NP, 16), jnp.float32),
    mesh=_vmesh,
    scratch_types=[
        pltpu.VMEM((CHP, K), jnp.int32),
        pltpu.VMEM((K, 16), jnp.float32),
        pltpu.VMEM_SHARED((NP, 16), jnp.float32),
    ],
    compiler_params=_sc_params,
)
def _deg(col_hbm, zero_hbm, out_hbm, colv, onesb, dacc):
    c = lax.axis_index("c")
    s = lax.axis_index("s")
    t = c * NS + s
    pltpu.sync_copy(zero_hbm.at[pl.ds(s * RPT, RPT)], dacc.at[pl.ds(s * RPT, RPT)])
    pltpu.sync_copy(col_hbm.at[t], colv)

    @pl.loop(0, K)
    def _(i):
        onesb[i, :] = jnp.ones((16,), jnp.float32)

    plsc.subcore_barrier()
    nch = jnp.where(c == 0, CH0, CH1)

    @pl.loop(0, nch)
    def _(j):
        pltpu.sync_copy(onesb, dacc.at[colv.at[j]], add=True)

    plsc.subcore_barrier()
    pltpu.sync_copy(dacc.at[pl.ds(s * RPT, RPT)],
                    out_hbm.at[c, pl.ds(s * RPT, RPT)])


def _mm_body(x_ref, w_ref, o_ref):
    o_ref[...] = jnp.dot(x_ref[...], w_ref[...], preferred_element_type=jnp.float32)


_mm1 = pl.pallas_call(
    _mm_body,
    grid=(G,),
    in_specs=[pl.BlockSpec((BLK, DI), lambda i: (i, 0)),
              pl.BlockSpec((DI, DH), lambda i: (0, 0))],
    out_specs=pl.BlockSpec((BLK, DH), lambda i: (i, 0)),
    out_shape=jax.ShapeDtypeStruct((NP, DH), jnp.float32),
)


def _dishp_body(dp_ref, xw_ref, dis_ref, hp_ref):
    deg = dp_ref[0, :, 0:1] + dp_ref[1, :, 0:1] + 1.0
    dis = lax.rsqrt(deg)
    dis_ref[...] = dis
    hp_ref[...] = dis * xw_ref[...]


_dishp = pl.pallas_call(
    _dishp_body,
    out_shape=(jax.ShapeDtypeStruct((NP, 1), jnp.float32),
               jax.ShapeDtypeStruct((NP, DH), jnp.float32)),
)


def _layer_body(acc_ref, hp_ref, dis_ref, b_ref, w_ref, o_ref):
    pre = dis_ref[...] * (acc_ref[0] + acc_ref[1] + hp_ref[...]) + b_ref[...]
    h = jnp.maximum(pre, 0.0)
    o_ref[...] = dis_ref[...] * jnp.dot(h, w_ref[...], preferred_element_type=jnp.float32)


_layer = pl.pallas_call(
    _layer_body,
    grid=(G,),
    in_specs=[pl.BlockSpec((NC, BLK, DH), lambda i: (0, i, 0)),
              pl.BlockSpec((BLK, DH), lambda i: (i, 0)),
              pl.BlockSpec((BLK, 1), lambda i: (i, 0)),
              pl.BlockSpec((1, DH), lambda i: (0, 0)),
              pl.BlockSpec((DH, DH), lambda i: (0, 0))],
    out_specs=pl.BlockSpec((BLK, DH), lambda i: (i, 0)),
    out_shape=jax.ShapeDtypeStruct((NP, DH), jnp.float32),
)


_layer5 = pl.pallas_call(
    _layer_body,
    grid=(G,),
    in_specs=[pl.BlockSpec((NC, BLK, DH), lambda i: (0, i, 0)),
              pl.BlockSpec((BLK, DH), lambda i: (i, 0)),
              pl.BlockSpec((BLK, 1), lambda i: (i, 0)),
              pl.BlockSpec((1, DH), lambda i: (0, 0)),
              pl.BlockSpec((DH, 32), lambda i: (0, 0))],
    out_specs=pl.BlockSpec((BLK, 32), lambda i: (i, 0)),
    out_shape=jax.ShapeDtypeStruct((NP, 32), jnp.float32),
)


def _head1_body(acc_ref, hp_ref, dis_ref, b5_ref, batch_ref,
                h5_ref, sums_ref, cnt_ref):
    i = pl.program_id(0)
    outsp = dis_ref[...] * (acc_ref[0] + acc_ref[1] + hp_ref[...])
    h5 = outsp[:, :DL] + b5_ref[...]
    h5_ref[...] = h5
    onehot = (batch_ref[...] == lax.broadcasted_iota(jnp.int32, (1, B), 1)
              ).astype(jnp.float32)
    ps = lax.dot_general(onehot, h5, (((0,), (0,)), ((), ())),
                         preferred_element_type=jnp.float32, precision=lax.Precision.HIGHEST)
    pc = lax.dot_general(onehot, jnp.ones((BLK, 1), jnp.float32),
                         (((0,), (0,)), ((), ())),
                         preferred_element_type=jnp.float32, precision=lax.Precision.HIGHEST)

    @pl.when(i == 0)
    def _():
        sums_ref[...] = jnp.zeros_like(sums_ref)
        cnt_ref[...] = jnp.zeros_like(cnt_ref)

    sums_ref[...] += ps
    cnt_ref[...] += pc


_head1 = pl.pallas_call(
    _head1_body,
    grid=(G,),
    in_specs=[pl.BlockSpec((NC, BLK, 32), lambda i: (0, i, 0)),
              pl.BlockSpec((BLK, 32), lambda i: (i, 0)),
              pl.BlockSpec((BLK, 1), lambda i: (i, 0)),
              pl.BlockSpec((1, DL), lambda i: (0, 0)),
              pl.BlockSpec((BLK, 1), lambda i: (i, 0))],
    out_specs=(pl.BlockSpec((BLK, DL), lambda i: (i, 0)),
               pl.BlockSpec((B, DL), lambda i: (0, 0)),
               pl.BlockSpec((B, 1), lambda i: (0, 0))),
    out_shape=(jax.ShapeDtypeStruct((NP, DL), jnp.float32),
               jax.ShapeDtypeStruct((B, DL), jnp.float32),
               jax.ShapeDtypeStruct((B, 1), jnp.float32)),
)


def _head2_body(h5_ref, sums_ref, cntv_ref, cnts_ref, base_ref,
                wl1_ref, bl1_ref, wl2_ref, bl2_ref, o_ref):
    rows = []
    off = jnp.int32(0)
    for b in range(B):
        pieces = []
        for l in range(LQ):
            bs = base_ref[b, l]
            idxv = jnp.clip(off + bs, 0, N - 1)
            rv = h5_ref[pl.ds(idxv, 1), :]
            rv = jnp.where(bs != 0, rv, 0.0)
            pieces.append(rv)
        rows.append(jnp.concatenate(pieces, axis=1))
        off = off + cnts_ref[b, 0].astype(jnp.int32)
    xs = jnp.concatenate(rows, axis=0)
    xg = sums_ref[...] / jnp.maximum(cntv_ref[...], 1.0)
    z = jnp.concatenate([xs, xg], axis=1)
    z = jnp.maximum(
        jnp.dot(z, wl1_ref[...], preferred_element_type=jnp.float32) + bl1_ref[...],
        0.0)
    o_ref[...] = jnp.dot(z, wl2_ref[...], preferred_element_type=jnp.float32) + bl2_ref[...]


_head2 = pl.pallas_call(
    _head2_body,
    in_specs=[pl.BlockSpec(memory_space=pltpu.VMEM),
              pl.BlockSpec(memory_space=pltpu.VMEM),
              pl.BlockSpec(memory_space=pltpu.VMEM),
              pl.BlockSpec(memory_space=pltpu.SMEM),
              pl.BlockSpec(memory_space=pltpu.SMEM),
              pl.BlockSpec(memory_space=pltpu.VMEM),
              pl.BlockSpec(memory_space=pltpu.VMEM),
              pl.BlockSpec(memory_space=pltpu.VMEM),
              pl.BlockSpec(memory_space=pltpu.VMEM)],
    out_shape=jax.ShapeDtypeStruct((B, 1), jnp.float32),
)


def kernel(x, edge_index, batch, base, W1, b1, W2, b2, W3, b3, W4, b4,
           W5, b5, Wl1, bl1, Wl2, bl2):
    def _pack(a):
        e0 = NS * CH0 * K
        c0 = a[:e0].reshape(NS, CH0, K)
        pad1 = jnp.full((NS * CH1 * K - (E - e0),), N, jnp.int32)
        c1 = jnp.concatenate([a[e0:], pad1]).reshape(NS, CH1, K)
        out = jnp.full((NW, CHP, K), N, jnp.int32)
        out = out.at[:NS, :CH0].set(c0)
        out = out.at[NS:, :CH1].set(c1)
        return out

    rowp = _pack(edge_index[0])
    colp = _pack(edge_index[1])
    xp = jnp.pad(x.astype(jnp.float32), ((0, NP - N), (0, 0)))
    batchp = jnp.pad(batch, (0, NP - N), constant_values=-1)[:, None]
    z64 = jnp.zeros((NP, DH), jnp.float32)
    z32 = jnp.zeros((NP, 32), jnp.float32)
    z16 = jnp.zeros((NP, 16), jnp.float32)
    W5p = jnp.pad(W5, ((0, 0), (0, 32 - DL)))

    xw = _mm1(xp, W1)
    degparts = _deg(colp, z16)
    dis, hp = _dishp(degparts, xw)

    for bb, Wn in ((b1, W2), (b2, W3), (b3, W4)):
        acc = _spmm(hp, rowp, colp, z64)
        hp = _layer(acc, hp, dis, bb[None, :], Wn)
    acc = _spmm(hp, rowp, colp, z64)
    hp = _layer5(acc, hp, dis, b4[None, :], W5p)
    acc = _spmm32(hp, rowp, colp, z32)
    h5, sums, cnt = _head1(acc, hp, dis, b5[None, :], batchp)
    out = _head2(h5, sums, cnt, cnt, base, Wl1, bl1[None, :], Wl2,
                 bl2[None, :])
    return out
